# Initial kernel scaffold; baseline (speedup 1.0000x reference)
import jax
import jax.numpy as jnp
from jax.experimental import pallas as pl


def kernel(x):
    raise NotImplementedError("write your pallas kernel here")


# speedup vs baseline: 9.3452x; 9.3452x over previous
import jax
import jax.numpy as jnp
from jax.experimental import pallas as pl
from jax.experimental.pallas import tpu as pltpu

K_CL = 512
N_TOK = 16384
D = 64
BLK = 512
NBLK = N_TOK // BLK
MAX_ITERS = 100
RTOL = 1e-4
ATOL = 1e-8


def _kernel_body(x_ref, c0_ref, loss_ref,
                 c_ref, sums_ref, counts_ref, ids_ref, conv_ref):
    c_ref[...] = c0_ref[...]
    conv_ref[0] = 0

    lane_iota = jax.lax.broadcasted_iota(jnp.int32, (BLK, K_CL), 1)
    ones_d = jnp.ones((1, D), jnp.float32)
    ones_blk = jnp.ones((BLK, 1), jnp.float32)

    def iter_body(_, carry):
        @pl.when(conv_ref[0] == 0)
        def _():
            c = c_ref[...]
            csq = c * c
            c2 = jax.lax.dot_general(
                ones_d, csq, (((1,), (1,)), ((), ())),
                preferred_element_type=jnp.float32)
            sums_ref[...] = jnp.zeros_like(sums_ref)
            counts_ref[...] = jnp.zeros_like(counts_ref)

            def blk_body(b, carry2):
                xb = x_ref[pl.ds(b * BLK, BLK), :]
                x2 = jnp.sum(xb * xb, axis=1, keepdims=True)
                dots = jax.lax.dot_general(
                    xb, c, (((1,), (1,)), ((), ())),
                    preferred_element_type=jnp.float32)
                dist = jnp.sqrt(jnp.maximum(x2 + c2 - 2.0 * dots, 0.0))
                dmin = jnp.min(dist, axis=1, keepdims=True)
                idcol = jnp.min(
                    jnp.where(dist == dmin, lane_iota, K_CL),
                    axis=1, keepdims=True).astype(jnp.int32)
                ids_ref[pl.ds(b * BLK, BLK), :] = idcol
                oh = (idcol == lane_iota).astype(jnp.float32)
                sums_ref[...] += jax.lax.dot_general(
                    oh, xb, (((0,), (0,)), ((), ())),
                    preferred_element_type=jnp.float32)
                counts_ref[...] += jax.lax.dot_general(
                    oh, ones_blk, (((0,), (0,)), ((), ())),
                    preferred_element_type=jnp.float32)
                return carry2

            jax.lax.fori_loop(0, NBLK, blk_body, 0, unroll=False)

            cnt = counts_ref[...]
            new_c = sums_ref[...] / jnp.maximum(cnt, 1.0)
            new_c = jnp.where(cnt > 0.0, new_c, c)
            ac = jnp.all(jnp.abs(c - new_c) <= ATOL + RTOL * jnp.abs(new_c))

            @pl.when(jnp.logical_not(ac))
            def _():
                c_ref[...] = new_c

            conv_ref[0] = ac.astype(jnp.int32)

        return carry

    jax.lax.fori_loop(0, MAX_ITERS, iter_body, 0, unroll=False)

    c = c_ref[...]

    def loss_blk(b, acc):
        xb = x_ref[pl.ds(b * BLK, BLK), :]
        logits = jax.lax.dot_general(
            xb, c, (((1,), (1,)), ((), ())),
            preferred_element_type=jnp.float32)
        m = jnp.max(logits, axis=1, keepdims=True)
        lse = m + jnp.log(jnp.sum(jnp.exp(logits - m), axis=1, keepdims=True))
        idcol = ids_ref[pl.ds(b * BLK, BLK), :]
        oh = (idcol == lane_iota).astype(jnp.float32)
        lab = jnp.sum(logits * oh, axis=1, keepdims=True)
        return acc + jnp.sum(lse - lab)

    acc = jax.lax.fori_loop(0, NBLK, loss_blk, jnp.float32(0.0), unroll=False)
    loss_ref[...] = jnp.broadcast_to(acc / jnp.float32(N_TOK), (1, 1))


def _run(x, c0, interpret=False):
    out = pl.pallas_call(
        _kernel_body,
        out_shape=jax.ShapeDtypeStruct((1, 1), jnp.float32),
        in_specs=[
            pl.BlockSpec(memory_space=pltpu.VMEM),
            pl.BlockSpec(memory_space=pltpu.VMEM),
        ],
        out_specs=pl.BlockSpec(memory_space=pltpu.VMEM),
        scratch_shapes=[
            pltpu.VMEM((K_CL, D), jnp.float32),
            pltpu.VMEM((K_CL, D), jnp.float32),
            pltpu.VMEM((K_CL, 1), jnp.float32),
            pltpu.VMEM((N_TOK, 1), jnp.int32),
            pltpu.SMEM((1,), jnp.int32),
        ],
        interpret=interpret,
    )(x, c0)
    return out[0, 0]


def kernel(x):
    perm = jax.random.permutation(jax.random.key(42), N_TOK)
    c0 = x[perm[:K_CL]]
    return _run(x, c0)
